# Initial kernel scaffold; baseline (speedup 1.0000x reference)
import jax
import jax.numpy as jnp
from jax.experimental import pallas as pl


def kernel(x, edge_index, W1, b1, W2, b2):
    raise NotImplementedError("write your pallas kernel here")


# speedup vs baseline: 13.4999x; 13.4999x over previous
import functools

import jax
import jax.numpy as jnp
from jax import lax
from jax.experimental import pallas as pl
from jax.experimental.pallas import tpu as pltpu
from jax.experimental.pallas import tpu_sc as plsc

N_NODES = 10000
N_EDGES = 320000
IN_C = 128
HID = 128
OUT_C = 64

NC = 2
NS = 16
NW = NC * NS
CHUNK = 80
EPW = N_EDGES // NW
NCHUNK = EPW // CHUNK
NPAD = 10240
SLAB = NPAD // NS
NPAD1D = NPAD
SLAB1D = SLAB

_mesh = plsc.VectorSubcoreMesh(core_axis_name="c", subcore_axis_name="s")


@functools.partial(
    pl.kernel,
    out_type=jax.ShapeDtypeStruct((NC, NPAD1D), jnp.float32),
    mesh=_mesh,
    scratch_types=[
        pltpu.VMEM((CHUNK,), jnp.int32),
        pltpu.VMEM((CHUNK,), jnp.float32),
        pltpu.VMEM_SHARED((NPAD1D,), jnp.float32),
    ],
)
def _deg_kernel(dst_hbm, zeros_hbm, deg_out, idx_v, ones_v, deg_sh):
    c = lax.axis_index("c")
    s = lax.axis_index("s")
    pltpu.sync_copy(zeros_hbm.at[pl.ds(s * SLAB1D, SLAB1D)],
                    deg_sh.at[pl.ds(s * SLAB1D, SLAB1D)])
    for j in range(CHUNK // 16):
        ones_v[pl.ds(j * 16, 16)] = jnp.ones((16,), jnp.float32)
    plsc.subcore_barrier()
    base = (c * NS + s) * EPW

    def body(i, carry):
        off = base + i * CHUNK
        pltpu.sync_copy(dst_hbm.at[pl.ds(off, CHUNK)], idx_v)
        pltpu.sync_copy(ones_v, deg_sh.at[idx_v], add=True)
        return carry

    lax.fori_loop(0, NCHUNK, body, 0)
    plsc.subcore_barrier()
    pltpu.sync_copy(deg_sh.at[pl.ds(s * SLAB1D, SLAB1D)],
                    deg_out.at[c, pl.ds(s * SLAB1D, SLAB1D)])


def _make_agg_kernel(d_feat):
    @functools.partial(
        pl.kernel,
        out_type=jax.ShapeDtypeStruct((NC, NPAD, d_feat), jnp.float32),
        mesh=_mesh,
        scratch_types=[
            pltpu.VMEM((CHUNK,), jnp.int32),
            pltpu.VMEM((CHUNK,), jnp.int32),
            pltpu.VMEM((CHUNK, d_feat), jnp.float32),
            pltpu.VMEM_SHARED((NPAD, d_feat), jnp.float32),
            pltpu.SemaphoreType.DMA,
        ],
        compiler_params=pltpu.CompilerParams(use_tc_tiling_on_sc=False),
    )
    def agg_kernel(g_hbm, src_hbm, dst_hbm, zeros_hbm, acc_out,
                   sidx, didx, rows, acc_sh, sem):
        c = lax.axis_index("c")
        s = lax.axis_index("s")
        pltpu.sync_copy(zeros_hbm.at[pl.ds(s * SLAB, SLAB)],
                        acc_sh.at[pl.ds(s * SLAB, SLAB)])
        plsc.subcore_barrier()
        base = (c * NS + s) * EPW

        def body(i, carry):
            off = base + i * CHUNK
            pltpu.sync_copy(src_hbm.at[pl.ds(off, CHUNK)], sidx)
            pltpu.sync_copy(dst_hbm.at[pl.ds(off, CHUNK)], didx)
            pltpu.async_copy(g_hbm.at[sidx], rows, sem).wait()
            pltpu.sync_copy(rows, acc_sh.at[didx], add=True)
            return carry

        lax.fori_loop(0, NCHUNK, body, 0)
        plsc.subcore_barrier()
        pltpu.sync_copy(acc_sh.at[pl.ds(s * SLAB, SLAB)],
                        acc_out.at[c, pl.ds(s * SLAB, SLAB)])

    return agg_kernel


_agg128 = _make_agg_kernel(HID)
_agg64 = _make_agg_kernel(OUT_C)


BN = 1000


def _t1_body(x_ref, w_ref, dega_ref, degb_ref, g_ref, dis_ref):
    deg = dega_ref[...] + degb_ref[...] + 1.0
    dis = lax.rsqrt(deg)
    g_ref[...] = dis * jnp.dot(x_ref[...], w_ref[...],
                               preferred_element_type=jnp.float32)
    dis_ref[...] = dis


def _t2_body(acc_ref, g1_ref, dis_ref, b_ref, w_ref, g2_ref):
    dis = dis_ref[...]
    h = dis * (acc_ref[0] + acc_ref[1] + g1_ref[...]) + b_ref[...]
    h = jnp.maximum(h, 0.0)
    g2_ref[...] = dis * jnp.dot(h, w_ref[...],
                                preferred_element_type=jnp.float32)


def _t3_body(acc_ref, g2_ref, dis_ref, b_ref, out_ref):
    out_ref[...] = (dis_ref[...] * (acc_ref[0] + acc_ref[1] + g2_ref[...])
                    + b_ref[...])


def kernel(x, edge_index, W1, b1, W2, b2):
    src = edge_index[0].astype(jnp.int32)
    dst = edge_index[1].astype(jnp.int32)
    zeros1d = jnp.zeros((NPAD1D,), jnp.float32)
    zeros_h = jnp.zeros((NPAD, HID), jnp.float32)
    zeros_o = jnp.zeros((NPAD, OUT_C), jnp.float32)

    deg_parts = _deg_kernel(dst, zeros1d)
    dega = deg_parts[0, :N_NODES, None]
    degb = deg_parts[1, :N_NODES, None]

    grid = (N_NODES // BN,)
    g1, dis = pl.pallas_call(
        _t1_body,
        grid=grid,
        in_specs=[
            pl.BlockSpec((BN, IN_C), lambda i: (i, 0)),
            pl.BlockSpec((IN_C, HID), lambda i: (0, 0)),
            pl.BlockSpec((BN, 1), lambda i: (i, 0)),
            pl.BlockSpec((BN, 1), lambda i: (i, 0)),
        ],
        out_specs=[
            pl.BlockSpec((BN, HID), lambda i: (i, 0)),
            pl.BlockSpec((BN, 1), lambda i: (i, 0)),
        ],
        out_shape=[
            jax.ShapeDtypeStruct((N_NODES, HID), jnp.float32),
            jax.ShapeDtypeStruct((N_NODES, 1), jnp.float32),
        ],
    )(x, W1, dega, degb)

    acc1 = _agg128(g1, src, dst, zeros_h)[:, :N_NODES, :]

    g2 = pl.pallas_call(
        _t2_body,
        grid=grid,
        in_specs=[
            pl.BlockSpec((NC, BN, HID), lambda i: (0, i, 0)),
            pl.BlockSpec((BN, HID), lambda i: (i, 0)),
            pl.BlockSpec((BN, 1), lambda i: (i, 0)),
            pl.BlockSpec((1, HID), lambda i: (0, 0)),
            pl.BlockSpec((HID, OUT_C), lambda i: (0, 0)),
        ],
        out_specs=pl.BlockSpec((BN, OUT_C), lambda i: (i, 0)),
        out_shape=jax.ShapeDtypeStruct((N_NODES, OUT_C), jnp.float32),
    )(acc1, g1, dis, b1[None, :], W2)

    acc2 = _agg64(g2, src, dst, zeros_o)[:, :N_NODES, :]

    out = pl.pallas_call(
        _t3_body,
        grid=grid,
        in_specs=[
            pl.BlockSpec((NC, BN, OUT_C), lambda i: (0, i, 0)),
            pl.BlockSpec((BN, OUT_C), lambda i: (i, 0)),
            pl.BlockSpec((BN, 1), lambda i: (i, 0)),
            pl.BlockSpec((1, OUT_C), lambda i: (0, 0)),
        ],
        out_specs=pl.BlockSpec((BN, OUT_C), lambda i: (i, 0)),
        out_shape=jax.ShapeDtypeStruct((N_NODES, OUT_C), jnp.float32),
    )(acc2, g2, dis, b2[None, :])

    return out
